# initial kernel scaffold (unmeasured)
import jax
import jax.numpy as jnp
from jax import lax
from jax.experimental import pallas as pl
from jax.experimental.pallas import tpu as pltpu

N_DEV = 16
M = 4096
N = 2048
M_CH = M // N_DEV


def kernel(x, w_mat):
    def body(x_ref, w_ref, out_ref, comm_ref, send_sems, recv_sems, credit_sem):
        d = lax.axis_index("i")
        right = jnp.mod(d + 1, N_DEV)
        left = jnp.mod(d + N_DEV - 1, N_DEV)

        def rows(c):
            return pl.ds(c * M_CH, M_CH)

        for i in range(N_DEV):
            out_ref[rows(i), :] = jnp.dot(
                x_ref[rows(i), :], w_ref[...],
                preferred_element_type=jnp.float32,
            )

        comm_ref[0, :, :] = out_ref[rows(d), :]

        barrier_sem = pltpu.get_barrier_semaphore()
        for nbr in (left, right):
            pl.semaphore_signal(
                barrier_sem, inc=1,
                device_id=(nbr,), device_id_type=pl.DeviceIdType.MESH,
            )
        pl.semaphore_wait(barrier_sem, 2)

        n_hops = 2 * (N_DEV - 1)
        for h in range(n_hops):
            slot = h % 2
            rslot = (h + 1) % 2
            if h >= 2:
                pl.semaphore_wait(credit_sem, 1)
            rdma = pltpu.make_async_remote_copy(
                src_ref=comm_ref.at[slot],
                dst_ref=comm_ref.at[rslot],
                send_sem=send_sems.at[slot],
                recv_sem=recv_sems.at[rslot],
                device_id=(right,),
                device_id_type=pl.DeviceIdType.MESH,
            )
            rdma.start()
            rdma.wait()

            if h < N_DEV - 1:
                c = jnp.mod(d - 1 - h + 2 * N_DEV, N_DEV)
                acc = comm_ref[rslot, :, :] + out_ref[rows(c), :]
                comm_ref[rslot, :, :] = acc
                if h == N_DEV - 2:
                    out_ref[rows(c), :] = acc
            else:
                c = jnp.mod(d - h + N_DEV - 1 + 2 * N_DEV, N_DEV)
                out_ref[rows(c), :] = comm_ref[rslot, :, :]

            if h <= n_hops - 3:
                pl.semaphore_signal(
                    credit_sem, inc=1,
                    device_id=(left,), device_id_type=pl.DeviceIdType.MESH,
                )

        amax = jnp.float32(0.0)
        for i in range(N_DEV):
            amax = jnp.maximum(amax, jnp.max(jnp.abs(out_ref[rows(i), :])))
        scale = amax / 448.0
        for i in range(N_DEV):
            y = out_ref[rows(i), :]
            q = (y / scale).astype(jnp.float8_e4m3fn)
            out_ref[rows(i), :] = q.astype(jnp.float32) * scale

    return pl.pallas_call(
        body,
        out_shape=jax.ShapeDtypeStruct((M, N), jnp.float32),
        in_specs=[
            pl.BlockSpec(memory_space=pltpu.VMEM),
            pl.BlockSpec(memory_space=pltpu.VMEM),
        ],
        out_specs=pl.BlockSpec(memory_space=pltpu.VMEM),
        scratch_shapes=[
            pltpu.VMEM((2, M_CH, N), jnp.float32),
            pltpu.SemaphoreType.DMA((2,)),
            pltpu.SemaphoreType.DMA((2,)),
            pltpu.SemaphoreType.REGULAR,
        ],
        compiler_params=pltpu.CompilerParams(collective_id=0),
    )(x, w_mat)


# baseline (device time: 787546 ns/iter reference)
import jax
import jax.numpy as jnp
from jax import lax
from jax.experimental import pallas as pl
from jax.experimental.pallas import tpu as pltpu

N_DEV = 16
M = 4096
N = 2048
M_CH = M // N_DEV


def kernel(x, w_mat):
    def body(x_ref, w_ref, out_ref, comm_ref, send_sems, recv_sems, credit_sem):
        d = lax.axis_index("i")
        right = jnp.mod(d + 1, N_DEV)
        left = jnp.mod(d + N_DEV - 1, N_DEV)

        def rows(c):
            return pl.ds(c * M_CH, M_CH)

        for i in range(N_DEV):
            out_ref[rows(i), :] = jnp.dot(
                x_ref[rows(i), :], w_ref[...],
                preferred_element_type=jnp.float32,
            )

        comm_ref[0, :, :] = out_ref[rows(d), :]

        barrier_sem = pltpu.get_barrier_semaphore()
        for nbr in (left, right):
            pl.semaphore_signal(
                barrier_sem, inc=1,
                device_id=(nbr,), device_id_type=pl.DeviceIdType.MESH,
            )
        pl.semaphore_wait(barrier_sem, 2)

        n_hops = 2 * (N_DEV - 1)
        for h in range(n_hops):
            slot = h % 2
            rslot = (h + 1) % 2
            if h >= 2:
                pl.semaphore_wait(credit_sem, 1)
            rdma = pltpu.make_async_remote_copy(
                src_ref=comm_ref.at[slot],
                dst_ref=comm_ref.at[rslot],
                send_sem=send_sems.at[slot],
                recv_sem=recv_sems.at[rslot],
                device_id=(right,),
                device_id_type=pl.DeviceIdType.MESH,
            )
            rdma.start()
            rdma.wait()

            if h < N_DEV - 1:
                c = jnp.mod(d - 1 - h + 2 * N_DEV, N_DEV)
                acc = comm_ref[rslot, :, :] + out_ref[rows(c), :]
                comm_ref[rslot, :, :] = acc
                if h == N_DEV - 2:
                    out_ref[rows(c), :] = acc
            else:
                c = jnp.mod(d - h + N_DEV - 1 + 2 * N_DEV, N_DEV)
                out_ref[rows(c), :] = comm_ref[rslot, :, :]

            if h <= n_hops - 3:
                pl.semaphore_signal(
                    credit_sem, inc=1,
                    device_id=(left,), device_id_type=pl.DeviceIdType.MESH,
                )

        amax = jnp.float32(0.0)
        for i in range(N_DEV):
            amax = jnp.maximum(amax, jnp.max(jnp.abs(out_ref[rows(i), :])))
        scale = amax / 448.0
        for i in range(N_DEV):
            y = out_ref[rows(i), :]
            q = (y / scale).astype(jnp.float8_e4m3fn)
            out_ref[rows(i), :] = q.astype(jnp.float32) * scale

    return pl.pallas_call(
        body,
        out_shape=jax.ShapeDtypeStruct((M, N), jnp.float32),
        in_specs=[
            pl.BlockSpec(memory_space=pltpu.VMEM),
            pl.BlockSpec(memory_space=pltpu.VMEM),
        ],
        out_specs=pl.BlockSpec(memory_space=pltpu.VMEM),
        scratch_shapes=[
            pltpu.VMEM((2, M_CH, N), jnp.float32),
            pltpu.SemaphoreType.DMA((2,)),
            pltpu.SemaphoreType.DMA((2,)),
            pltpu.SemaphoreType.REGULAR,
        ],
        compiler_params=pltpu.CompilerParams(
            collective_id=0,
            vmem_limit_bytes=100 * 1024 * 1024,
        ),
    )(x, w_mat)


# device time: 514735 ns/iter; 1.5300x vs baseline; 1.5300x over previous
import jax
import jax.numpy as jnp
from jax import lax
from jax.experimental import pallas as pl
from jax.experimental.pallas import tpu as pltpu

N_DEV = 16
M = 4096
N = 2048
M_CH = M // N_DEV
N_HALF = N // 2


def kernel(x, w_mat):
    def body(x_ref, w_ref, out_ref,
             comm_r, comm_l, send_sems_r, recv_sems_r, send_sems_l,
             recv_sems_l, credit_r, credit_l):
        d = lax.axis_index("i")
        right = jnp.mod(d + 1, N_DEV)
        left = jnp.mod(d + N_DEV - 1, N_DEV)

        def rows(c):
            return pl.ds(c * M_CH, M_CH)

        for i in range(N_DEV):
            out_ref[rows(i), :] = jnp.dot(
                x_ref[rows(i), :], w_ref[...],
                preferred_element_type=jnp.float32,
                precision=lax.Precision.HIGHEST,
            )

        comm_r[0, :, :] = out_ref[rows(d), :N_HALF]
        comm_l[0, :, :] = out_ref[rows(d), N_HALF:]

        barrier_sem = pltpu.get_barrier_semaphore()
        for nbr in (left, right):
            pl.semaphore_signal(
                barrier_sem, inc=1,
                device_id=(nbr,), device_id_type=pl.DeviceIdType.MESH,
            )
        pl.semaphore_wait(barrier_sem, 2)

        n_hops = 2 * (N_DEV - 1)
        for h in range(n_hops):
            slot = h % 2
            rslot = (h + 1) % 2
            if h >= 2:
                pl.semaphore_wait(credit_r, 1)
                pl.semaphore_wait(credit_l, 1)
            rdma_r = pltpu.make_async_remote_copy(
                src_ref=comm_r.at[slot],
                dst_ref=comm_r.at[rslot],
                send_sem=send_sems_r.at[slot],
                recv_sem=recv_sems_r.at[rslot],
                device_id=(right,),
                device_id_type=pl.DeviceIdType.MESH,
            )
            rdma_l = pltpu.make_async_remote_copy(
                src_ref=comm_l.at[slot],
                dst_ref=comm_l.at[rslot],
                send_sem=send_sems_l.at[slot],
                recv_sem=recv_sems_l.at[rslot],
                device_id=(left,),
                device_id_type=pl.DeviceIdType.MESH,
            )
            rdma_r.start()
            rdma_l.start()
            rdma_r.wait()
            rdma_l.wait()

            if h < N_DEV - 1:
                cr = jnp.mod(d - 1 - h + 2 * N_DEV, N_DEV)
                cl = jnp.mod(d + 1 + h, N_DEV)
                acc_r = comm_r[rslot, :, :] + out_ref[rows(cr), :N_HALF]
                acc_l = comm_l[rslot, :, :] + out_ref[rows(cl), N_HALF:]
                comm_r[rslot, :, :] = acc_r
                comm_l[rslot, :, :] = acc_l
                if h == N_DEV - 2:
                    out_ref[rows(cr), :N_HALF] = acc_r
                    out_ref[rows(cl), N_HALF:] = acc_l
            else:
                cr = jnp.mod(d - h + N_DEV - 1 + 2 * N_DEV, N_DEV)
                cl = jnp.mod(d + h - N_DEV + 1, N_DEV)
                out_ref[rows(cr), :N_HALF] = comm_r[rslot, :, :]
                out_ref[rows(cl), N_HALF:] = comm_l[rslot, :, :]

            if h <= n_hops - 3:
                pl.semaphore_signal(
                    credit_r, inc=1,
                    device_id=(left,), device_id_type=pl.DeviceIdType.MESH,
                )
                pl.semaphore_signal(
                    credit_l, inc=1,
                    device_id=(right,), device_id_type=pl.DeviceIdType.MESH,
                )

        amax = jnp.float32(0.0)
        for i in range(N_DEV):
            amax = jnp.maximum(amax, jnp.max(jnp.abs(out_ref[rows(i), :])))
        scale = amax / 448.0
        for i in range(N_DEV):
            y = out_ref[rows(i), :]
            q = (y / scale).astype(jnp.float8_e4m3fn)
            out_ref[rows(i), :] = q.astype(jnp.float32) * scale

    return pl.pallas_call(
        body,
        out_shape=jax.ShapeDtypeStruct((M, N), jnp.float32),
        in_specs=[
            pl.BlockSpec(memory_space=pltpu.VMEM),
            pl.BlockSpec(memory_space=pltpu.VMEM),
        ],
        out_specs=pl.BlockSpec(memory_space=pltpu.VMEM),
        scratch_shapes=[
            pltpu.VMEM((2, M_CH, N_HALF), jnp.float32),
            pltpu.VMEM((2, M_CH, N_HALF), jnp.float32),
            pltpu.SemaphoreType.DMA((2,)),
            pltpu.SemaphoreType.DMA((2,)),
            pltpu.SemaphoreType.DMA((2,)),
            pltpu.SemaphoreType.DMA((2,)),
            pltpu.SemaphoreType.REGULAR,
            pltpu.SemaphoreType.REGULAR,
        ],
        compiler_params=pltpu.CompilerParams(
            collective_id=0,
            vmem_limit_bytes=100 * 1024 * 1024,
        ),
    )(x, w_mat)


# device time: 391371 ns/iter; 2.0123x vs baseline; 1.3152x over previous
import jax
import jax.numpy as jnp
from jax import lax
from jax.experimental import pallas as pl
from jax.experimental.pallas import tpu as pltpu

N_DEV = 16
M = 4096
N = 2048
M_CH = M // N_DEV
N_HALF = N // 2
N_SUB = N_HALF // 2
N_SLOT = 4
N_HOPS = 2 * (N_DEV - 1)


def kernel(x, w_mat):
    def body(x_ref, w_ref, out_ref,
             comm_r, comm_l, send_sems_r, recv_sems_r, send_sems_l,
             recv_sems_l, credit_r, credit_l):
        d = lax.axis_index("i")
        right = jnp.mod(d + 1, N_DEV)
        left = jnp.mod(d + N_DEV - 1, N_DEV)

        def rows(c):
            return pl.ds(c * M_CH, M_CH)

        def cols(ring, s):
            return pl.ds(ring * N_HALF + s * N_SUB, N_SUB)

        def gemm_half(c, ring):
            out_ref[rows(c), pl.ds(ring * N_HALF, N_HALF)] = jnp.dot(
                x_ref[rows(c), :],
                w_ref[:, pl.ds(ring * N_HALF, N_HALF)],
                preferred_element_type=jnp.float32,
                precision=lax.Precision.HIGHEST,
            )

        def desc(ring, s, slot, rslot, nbr):
            comm = (comm_r, comm_l)[ring]
            ssem = (send_sems_r, send_sems_l)[ring]
            rsem = (recv_sems_r, recv_sems_l)[ring]
            return pltpu.make_async_remote_copy(
                src_ref=comm.at[slot, s],
                dst_ref=comm.at[rslot, s],
                send_sem=ssem.at[slot, s],
                recv_sem=rsem.at[rslot, s],
                device_id=(nbr,),
                device_id_type=pl.DeviceIdType.MESH,
            )

        def send_desc(ring, s, g):
            nbr = right if ring == 0 else left
            return desc(ring, s, jnp.mod(g, N_SLOT), jnp.mod(g + 1, N_SLOT), nbr)

        gemm_half(d, 0)
        gemm_half(d, 1)
        gemm_half(left, 0)
        gemm_half(right, 1)
        for s in range(2):
            comm_r[0, s] = out_ref[rows(d), cols(0, s)]
            comm_l[0, s] = out_ref[rows(d), cols(1, s)]

        barrier_sem = pltpu.get_barrier_semaphore()
        for nbr in (left, right):
            pl.semaphore_signal(
                barrier_sem, inc=1,
                device_id=(nbr,), device_id_type=pl.DeviceIdType.MESH,
            )
        pl.semaphore_wait(barrier_sem, 2)

        for ring in range(2):
            for s in range(2):
                send_desc(ring, s, 0).start()

        def hop(h, amax):
            slot = jnp.mod(h, N_SLOT)
            rslot = jnp.mod(h + 1, N_SLOT)

            @pl.when(h >= 1)
            def _():
                for ring in range(2):
                    for s in range(2):
                        send_desc(ring, s, h - 1).wait_send()

            @pl.when(jnp.logical_and(h >= 2, h <= N_HOPS - 3))
            def _():
                for s in range(2):
                    pl.semaphore_signal(
                        credit_r.at[s], inc=1,
                        device_id=(left,), device_id_type=pl.DeviceIdType.MESH,
                    )
                    pl.semaphore_signal(
                        credit_l.at[s], inc=1,
                        device_id=(right,), device_id_type=pl.DeviceIdType.MESH,
                    )

            c_r = jnp.mod(d - 1 - h + 4 * N_DEV, N_DEV)
            c_l = jnp.mod(d + 1 + h, N_DEV)

            for s in range(2):
                for ring in range(2):
                    comm = (comm_r, comm_l)[ring]
                    credit = (credit_r, credit_l)[ring]
                    c_rs = c_r if ring == 0 else c_l
                    send_desc(ring, s, h).wait_recv()

                    @pl.when(h <= N_DEV - 2)
                    def _():
                        comm[rslot, s] = (
                            comm[rslot, s] + out_ref[rows(c_rs), cols(ring, s)]
                        )

                    @pl.when(h < N_HOPS - 1)
                    def _():
                        @pl.when(h + 1 >= N_SLOT)
                        def _():
                            pl.semaphore_wait(credit.at[s], 1)
                        send_desc(ring, s, h + 1).start()

            c_ag_r = jnp.mod(d - h + N_DEV - 1 + 4 * N_DEV, N_DEV)
            c_ag_l = jnp.mod(d + h - N_DEV + 1 + 4 * N_DEV, N_DEV)
            st_r = jnp.where(h <= N_DEV - 2, c_r, c_ag_r)
            st_l = jnp.where(h <= N_DEV - 2, c_l, c_ag_l)

            @pl.when(h >= N_DEV - 2)
            def _():
                for s in range(2):
                    out_ref[rows(st_r), cols(0, s)] = comm_r[rslot, s]
                    out_ref[rows(st_l), cols(1, s)] = comm_l[rslot, s]

            piece = jnp.maximum(
                jnp.max(jnp.abs(comm_r[rslot])), jnp.max(jnp.abs(comm_l[rslot]))
            )
            amax = jnp.where(h >= N_DEV - 2, jnp.maximum(amax, piece), amax)

            @pl.when(h <= N_DEV - 3)
            def _():
                gemm_half(jnp.mod(d - 2 - h + 4 * N_DEV, N_DEV), 0)
                gemm_half(jnp.mod(d + 2 + h, N_DEV), 1)

            return amax

        amax = lax.fori_loop(0, N_HOPS, hop, jnp.float32(0.0))

        for ring in range(2):
            for s in range(2):
                send_desc(ring, s, N_HOPS - 1).wait_send()

        scale = amax / 448.0
        for i in range(N_DEV):
            y = out_ref[rows(i), :]
            q = (y / scale).astype(jnp.float8_e4m3fn)
            out_ref[rows(i), :] = q.astype(jnp.float32) * scale

    return pl.pallas_call(
        body,
        out_shape=jax.ShapeDtypeStruct((M, N), jnp.float32),
        in_specs=[
            pl.BlockSpec(memory_space=pltpu.VMEM),
            pl.BlockSpec(memory_space=pltpu.VMEM),
        ],
        out_specs=pl.BlockSpec(memory_space=pltpu.VMEM),
        scratch_shapes=[
            pltpu.VMEM((N_SLOT, 2, M_CH, N_SUB), jnp.float32),
            pltpu.VMEM((N_SLOT, 2, M_CH, N_SUB), jnp.float32),
            pltpu.SemaphoreType.DMA((N_SLOT, 2)),
            pltpu.SemaphoreType.DMA((N_SLOT, 2)),
            pltpu.SemaphoreType.DMA((N_SLOT, 2)),
            pltpu.SemaphoreType.DMA((N_SLOT, 2)),
            pltpu.SemaphoreType.REGULAR((2,)),
            pltpu.SemaphoreType.REGULAR((2,)),
        ],
        compiler_params=pltpu.CompilerParams(
            collective_id=0,
            vmem_limit_bytes=100 * 1024 * 1024,
        ),
    )(x, w_mat)
